# baseline (speedup 1.0000x reference)
import functools

import jax
import jax.numpy as jnp
from jax import lax
from jax.experimental import pallas as pl
from jax.experimental.pallas import tpu as pltpu
from jax.experimental.pallas import tpu_sc as plsc

_B = 1024
_SEQ = 256
_DIM = 256

_NBG = 8
_NQ = 4
_QROWS = _SEQ // _NQ
_BPG = _B // _NBG
_RC = 32
_CPB = _QROWS // _RC
_NCH = _BPG * _CPB
_DEPTH = 8
_LEAD = 4


def _sc_body(x_hbm, pe_hbm, out_hbm, pe_v, *bufs_and_sems):
    bufs = bufs_and_sems[0:_DEPTH]
    sin = bufs_and_sems[_DEPTH:2 * _DEPTH]
    sout = bufs_and_sems[2 * _DEPTH:3 * _DEPTH]

    wid = lax.axis_index("s") * 2 + lax.axis_index("c")
    bg = lax.div(wid, _NQ)
    q = lax.rem(wid, _NQ)
    pltpu.sync_copy(pe_hbm.at[pl.ds(q * _QROWS, _QROWS)], pe_v)

    def row0_of(t):
        b = bg * _BPG + lax.div(t, _CPB)
        return b * _SEQ + q * _QROWS + lax.rem(t, _CPB) * _RC

    def fire_in(t, i):
        pltpu.make_async_copy(
            x_hbm.at[pl.ds(row0_of(t), _RC)], bufs[i], sin[i]).start()

    def fire_out(t, i):
        pltpu.make_async_copy(
            bufs[i], out_hbm.at[pl.ds(row0_of(t), _RC)], sout[i]).start()

    def drain_in(i):
        pltpu.make_async_copy(
            x_hbm.at[pl.ds(0, _RC)], bufs[i], sin[i]).wait()

    def drain_out(i):
        pltpu.make_async_copy(
            bufs[i], out_hbm.at[pl.ds(0, _RC)], sout[i]).wait()

    def compute(t, i):
        p0 = lax.rem(t, _CPB) * _RC
        buf = bufs[i]

        def row(r, c):
            for j in range(_DIM // 16):
                d = pl.ds(j * 16, 16)
                buf[r, d] = buf[r, d] + pe_v[p0 + r, d]
            return c

        lax.fori_loop(0, _RC, row, 0)

    for i in range(_DEPTH):
        fire_in(i, i)

    def body(k, carry):
        for i in range(_DEPTH):
            t = k * _DEPTH + i
            drain_in(i)
            compute(t, i)
            fire_out(t, i)
            j = (i + _LEAD) % _DEPTH
            if i < _LEAD:
                @pl.when(k > 0)
                def _():
                    drain_out(j)
                    fire_in(t + _LEAD, j)
            else:
                @pl.when(k < _NCH // _DEPTH - 1)
                def _():
                    drain_out(j)
                    fire_in(t + _LEAD, j)
        return carry

    lax.fori_loop(0, _NCH // _DEPTH, body, 0)
    for i in range(_DEPTH):
        drain_out(i)


def _sc_add(x2d, pe):
    kfn = functools.partial(
        pl.kernel,
        out_type=jax.ShapeDtypeStruct((_B * _SEQ, _DIM), jnp.float32),
        mesh=plsc.VectorSubcoreMesh(core_axis_name="c", subcore_axis_name="s"),
        scratch_types=(
            [pltpu.VMEM((_QROWS, _DIM), jnp.float32)]
            + [pltpu.VMEM((_RC, _DIM), jnp.float32) for _ in range(_DEPTH)]
            + [pltpu.SemaphoreType.DMA for _ in range(2 * _DEPTH)]
        ),
    )(_sc_body)
    return kfn(x2d, pe)


def kernel(x, pos_emb):
    B, S, D = x.shape
    pe = pos_emb[:S]
    return _sc_add(x.reshape(B * S, D), pe).reshape(B, S, D)
